# Initial kernel scaffold; baseline (speedup 1.0000x reference)
import jax
import jax.numpy as jnp
from jax.experimental import pallas as pl


def kernel(x, batch, t_int):
    raise NotImplementedError("write your pallas kernel here")


# speedup vs baseline: 33.9441x; 33.9441x over previous
import numpy as np
import jax
import jax.numpy as jnp
from jax.experimental import pallas as pl
from jax.experimental.pallas import tpu as pltpu

_TIMESTEPS = 500
_K = 4
_B = 16


def _cosine_alphas_bar_host(timesteps, s=0.008):
    steps = timesteps + 2
    x = np.linspace(0, steps, steps)
    ac = np.cos(0.5 * np.pi * ((x / steps) + s) / (1 + s)) ** 2
    ac = ac / ac[0]
    alphas_step = ac[1:] / ac[:-1]
    betas = np.clip(1.0 - alphas_step, 0.0, 0.9999)
    return np.exp(np.cumsum(np.log(1.0 - betas))).astype(np.float32)


_ALPHAS_BAR = _cosine_alphas_bar_host(_TIMESTEPS)


def _fused_kernel(alpha_ref, x_ref, g_ref, b_ref, prob_ref, noise_ref):
    xt = x_ref[...]
    gt = g_ref[...]
    b = b_ref[...]

    a = jnp.zeros(b.shape, jnp.float32)
    for i in range(_B):
        a = jnp.where(b == i, alpha_ref[i], a)

    q_off = (1.0 - a) * 0.25
    q_diag = a + q_off

    diag_t = q_diag * xt
    off_t = q_off * xt
    ri = jax.lax.broadcasted_iota(jnp.int32, xt.shape, 0)

    def term(j):
        return jnp.where(ri == j, diag_t[j:j + 1, :], off_t[j:j + 1, :])

    p = (term(0) + term(1)) + (term(2) + term(3))
    prob_ref[...] = p

    s = (p[0:1, :] + p[1:2, :]) + (p[2:3, :] + p[3:4, :])
    probs = p / s
    z = gt + jnp.log(probs + 1e-12)

    best = z[0:1, :]
    idx = jnp.zeros(b.shape, jnp.int32)
    for i in range(1, _K):
        zi = z[i:i + 1, :]
        better = zi > best
        idx = jnp.where(better, i, idx)
        best = jnp.where(better, zi, best)
    noise_ref[...] = (ri == idx).astype(jnp.float32)


def kernel(x, batch, t_int):
    n = x.shape[0]
    alphas_bar = jnp.asarray(_ALPHAS_BAR)
    t_float = t_int.astype(jnp.float32) / _TIMESTEPS
    t_idx = jnp.round(t_float * _TIMESTEPS).astype(jnp.int32).squeeze(-1)
    alpha = alphas_bar[t_idx]

    g = jax.random.gumbel(jax.random.key(1), (n, _K), jnp.float32)

    xt = x.T
    gt = g.T
    b2 = batch[None, :]

    prob_t, noise_t = pl.pallas_call(
        _fused_kernel,
        out_shape=[
            jax.ShapeDtypeStruct((_K, n), jnp.float32),
            jax.ShapeDtypeStruct((_K, n), jnp.float32),
        ],
        in_specs=[
            pl.BlockSpec(memory_space=pltpu.SMEM),
            pl.BlockSpec(memory_space=pltpu.VMEM),
            pl.BlockSpec(memory_space=pltpu.VMEM),
            pl.BlockSpec(memory_space=pltpu.VMEM),
        ],
        out_specs=[
            pl.BlockSpec(memory_space=pltpu.VMEM),
            pl.BlockSpec(memory_space=pltpu.VMEM),
        ],
    )(alpha, xt, gt, b2)

    return prob_t.T, noise_t.T
